# baseline (speedup 1.0000x reference)
import functools

import jax
import jax.numpy as jnp
from jax import lax
from jax.experimental import pallas as pl
from jax.experimental.pallas import tpu as pltpu, tpu_sc as plsc

_D = 128
_NVOX = _D * _D * _D
_NB = 2
_NTOT = _NB * _NVOX
_NW = 32
_PER_W = _NTOT // _NW
_CH = 2048
_NCH = _PER_W // _CH
_L = 16


def _dim_interp(coord_i, t):
    cf = coord_i.astype(jnp.float32)
    cl = jnp.clip(cf + t, 0.0, 127.0)
    i0 = cl.astype(jnp.int32)
    f0 = i0.astype(jnp.float32)
    d1 = jnp.minimum(f0 + 1.0, 127.0) - cl
    i1 = jnp.minimum(i0 + 1, 127)
    return i0, i1, d1


def _warp_body(img_hbm, trf_hbm, out_hbm, *s):
    tvA, tvB = s[0], s[1]
    idxA, idxB = s[2:10], s[10:18]
    valA, valB = s[18:26], s[26:34]
    wA, wB = s[34:37], s[37:40]
    outA, outB = s[40], s[41]
    semA, semB = s[42], s[43]

    cid = lax.axis_index("c")
    sid = lax.axis_index("s")
    wid = sid * 2 + cid
    g0 = wid * _PER_W
    batch = g0 >> 21
    boff = batch << 21
    e0 = g0 - boff
    iota = lax.iota(jnp.int32, _L)

    def p1_fire(ch, tv, idxs, ws, vals, sem):
        gch = g0 + ch * _CH
        ech = e0 + ch * _CH
        pltpu.sync_copy(trf_hbm.at[pl.ds(gch * 3, 3 * _CH)], tv)

        def pass1(v, c1):
            e = ech + v * _L + iota
            kk = e & 127
            jj = (e >> 7) & 127
            ii = (e >> 14) & 127
            b3 = v * (3 * _L)
            tx = plsc.load_gather(tv, [b3 + 3 * iota + 0])
            ty = plsc.load_gather(tv, [b3 + 3 * iota + 1])
            tz = plsc.load_gather(tv, [b3 + 3 * iota + 2])
            x0, x1, wx = _dim_interp(ii, tx)
            y0, y1, wy = _dim_interp(jj, ty)
            z0, z1, wz = _dim_interp(kk, tz)
            sl = pl.ds(v * _L, _L)
            ws[0][sl] = wx
            ws[1][sl] = wy
            ws[2][sl] = wz
            xa0 = boff + (x0 << 14)
            xa1 = boff + (x1 << 14)
            yb0 = y0 << 7
            yb1 = y1 << 7
            idxs[0][sl] = xa0 + yb0 + z0
            idxs[1][sl] = xa0 + yb0 + z1
            idxs[2][sl] = xa0 + yb1 + z0
            idxs[3][sl] = xa0 + yb1 + z1
            idxs[4][sl] = xa1 + yb0 + z0
            idxs[5][sl] = xa1 + yb0 + z1
            idxs[6][sl] = xa1 + yb1 + z0
            idxs[7][sl] = xa1 + yb1 + z1
            return c1

        lax.fori_loop(0, _CH // _L, pass1, 0)
        for c in range(8):
            pltpu.async_copy(img_hbm.at[idxs[c]], vals[c], sem)

    def drain_blend_store(ch, idxs, ws, vals, outv, sem):
        for c in range(8):
            pltpu.make_async_copy(img_hbm.at[idxs[c]], vals[c], sem).wait()

        def pass2(v, c2):
            sl = pl.ds(v * _L, _L)
            wx = ws[0][sl]
            wy = ws[1][sl]
            wz = ws[2][sl]
            ux = 1.0 - wx
            uy = 1.0 - wy
            uz = 1.0 - wz
            p00 = wx * wy
            p01 = wx * uy
            p10 = ux * wy
            p11 = ux * uy
            acc = (p00 * wz) * vals[0][sl]
            acc = acc + (p00 * uz) * vals[1][sl]
            acc = acc + (p01 * wz) * vals[2][sl]
            acc = acc + (p01 * uz) * vals[3][sl]
            acc = acc + (p10 * wz) * vals[4][sl]
            acc = acc + (p10 * uz) * vals[5][sl]
            acc = acc + (p11 * wz) * vals[6][sl]
            acc = acc + (p11 * uz) * vals[7][sl]
            outv[sl] = acc
            return c2

        lax.fori_loop(0, _CH // _L, pass2, 0)
        gch = g0 + ch * _CH
        pltpu.sync_copy(outv, out_hbm.at[pl.ds(gch, _CH)])

    p1_fire(0, tvA, idxA, wA, valA, semA)

    def step(t, carry):
        ch0 = 2 * t
        p1_fire(ch0 + 1, tvB, idxB, wB, valB, semB)
        drain_blend_store(ch0, idxA, wA, valA, outA, semA)

        @pl.when(ch0 + 2 < _NCH)
        def _():
            p1_fire(ch0 + 2, tvA, idxA, wA, valA, semA)

        drain_blend_store(ch0 + 1, idxB, wB, valB, outB, semB)
        return carry

    lax.fori_loop(0, _NCH // 2, step, 0)


_warp = functools.partial(
    pl.kernel,
    out_type=jax.ShapeDtypeStruct((_NTOT,), jnp.float32),
    mesh=plsc.VectorSubcoreMesh(core_axis_name="c", subcore_axis_name="s"),
    compiler_params=pltpu.CompilerParams(needs_layout_passes=False),
    scratch_types=(
        [pltpu.VMEM((3 * _CH,), jnp.float32) for _ in range(2)]
        + [pltpu.VMEM((_CH,), jnp.int32) for _ in range(16)]
        + [pltpu.VMEM((_CH,), jnp.float32) for _ in range(16)]
        + [pltpu.VMEM((_CH,), jnp.float32) for _ in range(6)]
        + [pltpu.VMEM((_CH,), jnp.float32) for _ in range(2)]
        + [pltpu.SemaphoreType.DMA, pltpu.SemaphoreType.DMA]
    ),
)(_warp_body)


@jax.jit
def kernel(img, trf):
    imgf = img.reshape(_NTOT)
    trff = trf.reshape(_NTOT * 3)
    out = _warp(imgf, trff)
    return out.reshape(img.shape)


# speedup vs baseline: 4.8942x; 4.8942x over previous
import functools

import jax
import jax.numpy as jnp
from jax import lax
from jax.experimental import pallas as pl
from jax.experimental.pallas import tpu as pltpu, tpu_sc as plsc

_D = 128
_NVOX = _D * _D * _D
_NB = 2
_NTOT = _NB * _NVOX
_NW = 32
_PER_W = _NTOT // _NW
_CH = 2048
_NCH = _PER_W // _CH
_L = 16


def _dim_interp(coord_i, t):
    cf = coord_i.astype(jnp.float32)
    cl = jnp.clip(cf + t, 0.0, 127.0)
    i0 = cl.astype(jnp.int32)
    f0 = i0.astype(jnp.float32)
    d1 = jnp.minimum(f0 + 1.0, 127.0) - cl
    i1 = jnp.minimum(i0 + 1, 127)
    return i0, i1, d1


def _warp_body(img_hbm, trf_hbm, out_hbm, *s):
    tvA, tvB = s[0:3], s[3:6]
    idxA, idxB = s[6:14], s[14:22]
    valA, valB = s[22:30], s[30:38]
    wA, wB = s[38:41], s[41:44]
    outA, outB = s[44], s[45]
    semA, semB = s[46], s[47]
    tsemA, tsemB = s[48], s[49]
    osemA, osemB = s[50], s[51]

    cid = lax.axis_index("c")
    sid = lax.axis_index("s")
    wid = sid * 2 + cid
    g0 = wid * _PER_W
    batch = g0 >> 21
    boff = batch << 21
    e0 = g0 - boff
    iota = lax.iota(jnp.int32, _L)

    def _t_slices(ch):
        prow = ((g0 + ch * _CH) >> 14) * 3
        poff = (e0 + ch * _CH) & 16383
        return [pl.multiple_of((prow + d) * 16384 + poff, 8) for d in range(3)]

    def t_start(ch, tv, tsem):
        del ch, tv, tsem

    def t_wait(ch, tv, tsem):
        del tsem
        for d, off in enumerate(_t_slices(ch)):
            pltpu.sync_copy(trf_hbm.at[pl.ds(off, _CH)], tv[d])

    def p1_fire(ch, tv, idxs, ws, vals, sem, tsem):
        ech = e0 + ch * _CH
        t_wait(ch, tv, tsem)

        def pass1(v, c1):
            e = ech + v * _L + iota
            kk = e & 127
            jj = (e >> 7) & 127
            ii = (e >> 14) & 127
            sl = pl.ds(v * _L, _L)
            x0, x1, wx = _dim_interp(ii, tv[0][sl])
            y0, y1, wy = _dim_interp(jj, tv[1][sl])
            z0, z1, wz = _dim_interp(kk, tv[2][sl])
            ws[0][sl] = wx
            ws[1][sl] = wy
            ws[2][sl] = wz
            xa0 = boff + (x0 << 14)
            xa1 = boff + (x1 << 14)
            yb0 = y0 << 7
            yb1 = y1 << 7
            idxs[0][sl] = xa0 + yb0 + z0
            idxs[1][sl] = xa0 + yb0 + z1
            idxs[2][sl] = xa0 + yb1 + z0
            idxs[3][sl] = xa0 + yb1 + z1
            idxs[4][sl] = xa1 + yb0 + z0
            idxs[5][sl] = xa1 + yb0 + z1
            idxs[6][sl] = xa1 + yb1 + z0
            idxs[7][sl] = xa1 + yb1 + z1
            return c1

        lax.fori_loop(0, _CH // _L, pass1, 0)
        for c in range(8):
            pltpu.async_copy(img_hbm.at[idxs[c]], vals[c], sem)

    def drain_blend_store(ch, idxs, ws, vals, outv, sem, osem):
        for c in range(8):
            pltpu.make_async_copy(img_hbm.at[idxs[c]], vals[c], sem).wait()

        @pl.when(ch >= 2)
        def _():
            gprev = g0 + (ch - 2) * _CH
            pltpu.make_async_copy(outv, out_hbm.at[pl.ds(gprev, _CH)],
                                  osem).wait()

        def pass2(v, c2):
            sl = pl.ds(v * _L, _L)
            wx = ws[0][sl]
            wy = ws[1][sl]
            wz = ws[2][sl]
            ux = 1.0 - wx
            uy = 1.0 - wy
            uz = 1.0 - wz
            p00 = wx * wy
            p01 = wx * uy
            p10 = ux * wy
            p11 = ux * uy
            acc = (p00 * wz) * vals[0][sl]
            acc = acc + (p00 * uz) * vals[1][sl]
            acc = acc + (p01 * wz) * vals[2][sl]
            acc = acc + (p01 * uz) * vals[3][sl]
            acc = acc + (p10 * wz) * vals[4][sl]
            acc = acc + (p10 * uz) * vals[5][sl]
            acc = acc + (p11 * wz) * vals[6][sl]
            acc = acc + (p11 * uz) * vals[7][sl]
            outv[sl] = acc
            return c2

        lax.fori_loop(0, _CH // _L, pass2, 0)
        gch = g0 + ch * _CH
        pltpu.async_copy(outv, out_hbm.at[pl.ds(gch, _CH)], osem)

    t_start(0, tvA, tsemA)
    t_start(1, tvB, tsemB)
    p1_fire(0, tvA, idxA, wA, valA, semA, tsemA)
    t_start(2, tvA, tsemA)

    def step(t, carry):
        ch0 = 2 * t
        p1_fire(ch0 + 1, tvB, idxB, wB, valB, semB, tsemB)

        @pl.when(ch0 + 3 < _NCH)
        def _():
            t_start(ch0 + 3, tvB, tsemB)

        drain_blend_store(ch0, idxA, wA, valA, outA, semA, osemA)

        @pl.when(ch0 + 2 < _NCH)
        def _():
            p1_fire(ch0 + 2, tvA, idxA, wA, valA, semA, tsemA)

            @pl.when(ch0 + 4 < _NCH)
            def _():
                t_start(ch0 + 4, tvA, tsemA)

        drain_blend_store(ch0 + 1, idxB, wB, valB, outB, semB, osemB)
        return carry

    lax.fori_loop(0, _NCH // 2, step, 0)

    pltpu.make_async_copy(outA, out_hbm.at[pl.ds(g0 + (_NCH - 2) * _CH, _CH)],
                          osemA).wait()
    pltpu.make_async_copy(outB, out_hbm.at[pl.ds(g0 + (_NCH - 1) * _CH, _CH)],
                          osemB).wait()


_warp = functools.partial(
    pl.kernel,
    out_type=jax.ShapeDtypeStruct((_NTOT,), jnp.float32),
    mesh=plsc.VectorSubcoreMesh(core_axis_name="c", subcore_axis_name="s"),
    compiler_params=pltpu.CompilerParams(needs_layout_passes=False),
    scratch_types=(
        [pltpu.VMEM((_CH,), jnp.float32) for _ in range(6)]
        + [pltpu.VMEM((_CH,), jnp.int32) for _ in range(16)]
        + [pltpu.VMEM((_CH,), jnp.float32) for _ in range(16)]
        + [pltpu.VMEM((_CH,), jnp.float32) for _ in range(6)]
        + [pltpu.VMEM((_CH,), jnp.float32) for _ in range(2)]
        + [pltpu.SemaphoreType.DMA for _ in range(6)]
    ),
)(_warp_body)


@jax.jit
def kernel(img, trf):
    imgf = img.reshape(_NTOT)
    trfp = trf.transpose(0, 1, 4, 2, 3).reshape(_NTOT * 3)
    out = _warp(imgf, trfp)
    return out.reshape(img.shape)


# speedup vs baseline: 4.9553x; 1.0125x over previous
import functools

import jax
import jax.numpy as jnp
from jax import lax
from jax.experimental import pallas as pl
from jax.experimental.pallas import tpu as pltpu, tpu_sc as plsc

_D = 128
_NVOX = _D * _D * _D
_NB = 2
_NTOT = _NB * _NVOX
_NW = 32
_PER_W = _NTOT // _NW
_CH = 2048
_NCH = _PER_W // _CH
_L = 16


def _dim_interp(coord_i, t):
    cf = coord_i.astype(jnp.float32)
    cl = jnp.clip(cf + t, 0.0, 127.0)
    i0 = cl.astype(jnp.int32)
    f0 = i0.astype(jnp.float32)
    d1 = jnp.minimum(f0 + 1.0, 127.0) - cl
    i1 = jnp.minimum(i0 + 1, 127)
    return i0, i1, d1


def _warp_body(img_hbm, trf_hbm, out_hbm, *s):
    tvA, tvB = s[0:3], s[3:6]
    idxA, idxB = s[6:14], s[14:22]
    valA, valB = s[22:30], s[30:38]
    wA, wB = s[38:41], s[41:44]
    outA, outB = s[44], s[45]
    semA, semB = s[46], s[47]
    tsemA, tsemB = s[48], s[49]
    osemA, osemB = s[50], s[51]

    cid = lax.axis_index("c")
    sid = lax.axis_index("s")
    wid = sid * 2 + cid
    g0 = wid * _PER_W
    batch = g0 >> 21
    boff = batch << 21
    e0 = g0 - boff
    iota = lax.iota(jnp.int32, _L)

    def _t_slices(ch):
        prow = ((g0 + ch * _CH) >> 14) * 3
        poff = (e0 + ch * _CH) & 16383
        return [pl.multiple_of((prow + d) * 16384 + poff, 8) for d in range(3)]

    def t_start(ch, tv, tsem):
        for d, off in enumerate(_t_slices(ch)):
            pltpu.async_copy(trf_hbm.at[pl.ds(off, _CH)], tv[d], tsem)

    def t_wait(ch, tv, tsem):
        for d, off in enumerate(_t_slices(ch)):
            pltpu.make_async_copy(trf_hbm.at[pl.ds(off, _CH)], tv[d],
                                  tsem).wait()

    def p1_fire(ch, tv, idxs, ws, vals, sem, tsem):
        ech = e0 + ch * _CH
        t_wait(ch, tv, tsem)

        def pass1(v, c1):
            e = ech + v * _L + iota
            kk = e & 127
            jj = (e >> 7) & 127
            ii = (e >> 14) & 127
            sl = pl.ds(v * _L, _L)
            x0, x1, wx = _dim_interp(ii, tv[0][sl])
            y0, y1, wy = _dim_interp(jj, tv[1][sl])
            z0, z1, wz = _dim_interp(kk, tv[2][sl])
            ws[0][sl] = wx
            ws[1][sl] = wy
            ws[2][sl] = wz
            xa0 = boff + (x0 << 14)
            xa1 = boff + (x1 << 14)
            yb0 = y0 << 7
            yb1 = y1 << 7
            idxs[0][sl] = xa0 + yb0 + z0
            idxs[1][sl] = xa0 + yb0 + z1
            idxs[2][sl] = xa0 + yb1 + z0
            idxs[3][sl] = xa0 + yb1 + z1
            idxs[4][sl] = xa1 + yb0 + z0
            idxs[5][sl] = xa1 + yb0 + z1
            idxs[6][sl] = xa1 + yb1 + z0
            idxs[7][sl] = xa1 + yb1 + z1
            return c1

        lax.fori_loop(0, _CH // _L, pass1, 0)
        for c in range(8):
            pltpu.async_copy(img_hbm.at[idxs[c]], vals[c], sem)

    def drain_blend_store(ch, idxs, ws, vals, outv, sem, osem):
        for c in range(8):
            pltpu.make_async_copy(img_hbm.at[idxs[c]], vals[c], sem).wait()

        @pl.when(ch >= 2)
        def _():
            gprev = g0 + (ch - 2) * _CH
            pltpu.make_async_copy(outv, out_hbm.at[pl.ds(gprev, _CH)],
                                  osem).wait()

        def pass2(v, c2):
            sl = pl.ds(v * _L, _L)
            wx = ws[0][sl]
            wy = ws[1][sl]
            wz = ws[2][sl]
            ux = 1.0 - wx
            uy = 1.0 - wy
            uz = 1.0 - wz
            p00 = wx * wy
            p01 = wx * uy
            p10 = ux * wy
            p11 = ux * uy
            acc = (p00 * wz) * vals[0][sl]
            acc = acc + (p00 * uz) * vals[1][sl]
            acc = acc + (p01 * wz) * vals[2][sl]
            acc = acc + (p01 * uz) * vals[3][sl]
            acc = acc + (p10 * wz) * vals[4][sl]
            acc = acc + (p10 * uz) * vals[5][sl]
            acc = acc + (p11 * wz) * vals[6][sl]
            acc = acc + (p11 * uz) * vals[7][sl]
            outv[sl] = acc
            return c2

        lax.fori_loop(0, _CH // _L, pass2, 0)
        gch = g0 + ch * _CH
        pltpu.async_copy(outv, out_hbm.at[pl.ds(gch, _CH)], osem)

    t_start(0, tvA, tsemA)
    t_start(1, tvB, tsemB)
    p1_fire(0, tvA, idxA, wA, valA, semA, tsemA)
    t_start(2, tvA, tsemA)

    def step(t, carry):
        ch0 = 2 * t
        p1_fire(ch0 + 1, tvB, idxB, wB, valB, semB, tsemB)

        @pl.when(ch0 + 3 < _NCH)
        def _():
            t_start(ch0 + 3, tvB, tsemB)

        drain_blend_store(ch0, idxA, wA, valA, outA, semA, osemA)

        @pl.when(ch0 + 2 < _NCH)
        def _():
            p1_fire(ch0 + 2, tvA, idxA, wA, valA, semA, tsemA)

            @pl.when(ch0 + 4 < _NCH)
            def _():
                t_start(ch0 + 4, tvA, tsemA)

        drain_blend_store(ch0 + 1, idxB, wB, valB, outB, semB, osemB)
        return carry

    lax.fori_loop(0, _NCH // 2, step, 0)

    pltpu.make_async_copy(outA, out_hbm.at[pl.ds(g0 + (_NCH - 2) * _CH, _CH)],
                          osemA).wait()
    pltpu.make_async_copy(outB, out_hbm.at[pl.ds(g0 + (_NCH - 1) * _CH, _CH)],
                          osemB).wait()


_warp = functools.partial(
    pl.kernel,
    out_type=jax.ShapeDtypeStruct((_NTOT,), jnp.float32),
    mesh=plsc.VectorSubcoreMesh(core_axis_name="c", subcore_axis_name="s"),
    compiler_params=pltpu.CompilerParams(needs_layout_passes=False),
    scratch_types=(
        [pltpu.VMEM((_CH,), jnp.float32) for _ in range(6)]
        + [pltpu.VMEM((_CH,), jnp.int32) for _ in range(16)]
        + [pltpu.VMEM((_CH,), jnp.float32) for _ in range(16)]
        + [pltpu.VMEM((_CH,), jnp.float32) for _ in range(6)]
        + [pltpu.VMEM((_CH,), jnp.float32) for _ in range(2)]
        + [pltpu.SemaphoreType.DMA for _ in range(6)]
    ),
)(_warp_body)


@jax.jit
def kernel(img, trf):
    imgf = img.reshape(_NTOT)
    trfp = trf.transpose(0, 1, 4, 2, 3).reshape(_NTOT * 3)
    out = _warp(imgf, trfp)
    return out.reshape(img.shape)
